# baseline (speedup 1.0000x reference)
import functools

import jax
import jax.numpy as jnp
from jax import lax
from jax.experimental import pallas as pl
from jax.experimental.pallas import tpu as pltpu
from jax.experimental.pallas import tpu_sc as plsc

N = 10000
D = 128
E = 320000
NS = 16
CH = 128
CPT = 160
TOTCH = NS * CPT
EPAD = TOTCH * CH
PH1 = 128
PH2 = CPT - PH1
NPAD = 10112
RPT = NPAD // NS
DEGR = 80
DRPT = 8

f32 = jnp.float32
i32 = jnp.int32


@functools.lru_cache(maxsize=None)
def _mesh():
    return plsc.VectorSubcoreMesh(
        core_axis_name="c", subcore_axis_name="s", num_cores=1, num_subcores=NS
    )


def _zero_vmem_rows(ref, nrows):
    zero16 = jnp.zeros((16,), f32)

    def zrow(i, c):
        for k in range(8):
            ref[i, pl.ds(k * 16, 16)] = zero16
        return c

    lax.fori_loop(0, nrows, zrow, 0)


def _deg_body(ep_hbm, deg_out, didx, deg1d, deg2d, rowidx, shared_deg):
    sid = lax.axis_index("s")

    zero16 = jnp.zeros((16,), f32)

    def z1(i, c):
        deg1d[pl.ds(i * 16, 16)] = zero16
        return c

    lax.fori_loop(0, DEGR * D // 16, z1, 0)
    _zero_vmem_rows(deg2d, DEGR)

    @pl.when(sid == 0)
    def _():
        pltpu.sync_copy(deg2d, shared_deg)

    for k in range(DEGR // 16):
        rowidx[0, pl.ds(k * 16, 16)] = lax.iota(i32, 16) + (16 * k)

    pltpu.sync_copy(ep_hbm.at[pl.ds(sid * CPT, CPT)], didx)
    plsc.subcore_barrier()

    ones16 = jnp.ones((16,), f32)

    def jbody(j, c):
        for k in range(CH // 16):
            v = didx[j, pl.ds(k * 16, 16)]
            plsc.addupdate_scatter(
                deg1d, [lax.shift_right_logical(v, 16)], ones16
            )
        return c

    lax.fori_loop(0, CPT, jbody, 0)

    def stage(r, c):
        for k in range(8):
            deg2d[r, pl.ds(k * 16, 16)] = deg1d[pl.ds(r * D + k * 16, 16)]
        return c

    lax.fori_loop(0, DEGR, stage, 0)

    pltpu.sync_copy(deg2d, shared_deg.at[rowidx.at[0]], add=True)
    plsc.subcore_barrier()

    @pl.when(sid < DEGR // DRPT)
    def _():
        r0 = sid * DRPT
        pltpu.sync_copy(
            shared_deg.at[pl.ds(r0, DRPT)], deg_out.at[pl.ds(r0, DRPT)]
        )


@functools.lru_cache(maxsize=None)
def _deg_kernel():
    return pl.kernel(
        _deg_body,
        out_type=jax.ShapeDtypeStruct((DEGR, D), f32),
        mesh=_mesh(),
        compiler_params=pltpu.CompilerParams(needs_layout_passes=False),
        scratch_types=[
            pltpu.VMEM((CPT, CH), i32),
            pltpu.VMEM((DEGR * D,), f32),
            pltpu.VMEM((DEGR, D), f32),
            pltpu.VMEM((1, DEGR), i32),
            pltpu.VMEM_SHARED((DEGR, D), f32),
        ],
    )


def _agg_body(g_hbm, ep_hbm, acc_out, eidx, rbuf0, rbuf1, sstage, dstage,
              gsem0, gsem1, shared_acc):
    sid = lax.axis_index("s")

    _zero_vmem_rows(rbuf0, CH)

    base = sid * RPT
    nfull = RPT // CH
    for b in range(nfull):
        pltpu.sync_copy(rbuf0, shared_acc.at[pl.ds(base + b * CH, CH)])
    rem = RPT - nfull * CH
    pltpu.sync_copy(
        rbuf0.at[pl.ds(0, rem)], shared_acc.at[pl.ds(base + nfull * CH, rem)]
    )

    plsc.subcore_barrier()

    def unpack(j, b):
        for k in range(CH // 16):
            v = eidx[j, pl.ds(k * 16, 16)]
            sstage[b, pl.ds(k * 16, 16)] = lax.bitwise_and(v, 0xFFFF)
            dstage[b, pl.ds(k * 16, 16)] = lax.shift_right_logical(v, 16)

    def run(chunk0, cnt):
        pltpu.sync_copy(
            ep_hbm.at[pl.ds(chunk0, cnt)], eidx.at[pl.ds(0, cnt)]
        )
        unpack(0, 0)
        unpack(1, 1)
        pltpu.async_copy(g_hbm.at[sstage.at[0]], rbuf0, gsem0)
        pltpu.async_copy(g_hbm.at[sstage.at[1]], rbuf1, gsem1)

        def jbody(jj, c):
            j = 2 * jj
            pltpu.make_async_copy(g_hbm.at[sstage.at[0]], rbuf0, gsem0).wait()
            pltpu.sync_copy(rbuf0, shared_acc.at[dstage.at[0]], add=True)
            unpack(j + 2, 0)
            pltpu.async_copy(g_hbm.at[sstage.at[0]], rbuf0, gsem0)
            pltpu.make_async_copy(g_hbm.at[sstage.at[1]], rbuf1, gsem1).wait()
            pltpu.sync_copy(rbuf1, shared_acc.at[dstage.at[1]], add=True)
            unpack(j + 3, 1)
            pltpu.async_copy(g_hbm.at[sstage.at[1]], rbuf1, gsem1)
            return c

        lax.fori_loop(0, cnt // 2 - 1, jbody, 0)

        pltpu.make_async_copy(g_hbm.at[sstage.at[0]], rbuf0, gsem0).wait()
        pltpu.sync_copy(rbuf0, shared_acc.at[dstage.at[0]], add=True)
        pltpu.make_async_copy(g_hbm.at[sstage.at[1]], rbuf1, gsem1).wait()
        pltpu.sync_copy(rbuf1, shared_acc.at[dstage.at[1]], add=True)

    run(sid * CPT, PH1)
    run(sid * CPT + PH1, PH2)

    plsc.subcore_barrier()
    pltpu.sync_copy(
        shared_acc.at[pl.ds(base, RPT)], acc_out.at[pl.ds(base, RPT)]
    )


@functools.lru_cache(maxsize=None)
def _agg_kernel():
    return pl.kernel(
        _agg_body,
        out_type=jax.ShapeDtypeStruct((NPAD, D), f32),
        mesh=_mesh(),
        compiler_params=pltpu.CompilerParams(needs_layout_passes=False),
        scratch_types=[
            pltpu.VMEM((PH1, CH), i32),
            pltpu.VMEM((CH, D), f32),
            pltpu.VMEM((CH, D), f32),
            pltpu.VMEM((2, CH), i32),
            pltpu.VMEM((2, CH), i32),
            pltpu.SemaphoreType.DMA,
            pltpu.SemaphoreType.DMA,
            pltpu.VMEM_SHARED((NPAD, D), f32),
        ],
    )


_BLK = 400
_GRID = N // _BLK


def _dis_body(deg_ref, dis_ref):
    dis_ref[...] = lax.rsqrt(1.0 + deg_ref[...])


def _dis_call(deg):
    return pl.pallas_call(
        _dis_body,
        out_shape=jax.ShapeDtypeStruct((DEGR, D), f32),
    )(deg)


def _lin1_body(x_ref, w_ref, b_ref, dis_ref, g_ref, hd_ref):
    h = jnp.dot(x_ref[...], w_ref[...], preferred_element_type=f32)
    h = h + b_ref[...]
    dis = dis_ref[...]
    g_ref[...] = h * dis
    hd_ref[...] = h * (dis * dis)


def _lin1_call(x, W1, b1, dis_col):
    return pl.pallas_call(
        _lin1_body,
        grid=(_GRID,),
        in_specs=[
            pl.BlockSpec((_BLK, D), lambda i: (i, 0)),
            pl.BlockSpec((D, D), lambda i: (0, 0)),
            pl.BlockSpec((1, D), lambda i: (0, 0)),
            pl.BlockSpec((_BLK, 1), lambda i: (i, 0)),
        ],
        out_specs=[
            pl.BlockSpec((_BLK, D), lambda i: (i, 0)),
            pl.BlockSpec((_BLK, D), lambda i: (i, 0)),
        ],
        out_shape=[
            jax.ShapeDtypeStruct((N, D), f32),
            jax.ShapeDtypeStruct((N, D), f32),
        ],
    )(x, W1, b1, dis_col)


def _elu(s):
    return jnp.where(s > 0, s, jnp.exp(s) - 1.0)


def _lin2_body(a_ref, hd_ref, dis_ref, w_ref, b_ref, g_ref, hd2_ref):
    dis = dis_ref[...]
    s = a_ref[...] * dis + hd_ref[...]
    o = _elu(s)
    h2 = jnp.dot(o, w_ref[...], preferred_element_type=f32) + b_ref[...]
    g_ref[...] = h2 * dis
    hd2_ref[...] = h2 * (dis * dis)


def _lin2_call(acc1, hd1, dis_col, W2, b2):
    return pl.pallas_call(
        _lin2_body,
        grid=(_GRID,),
        in_specs=[
            pl.BlockSpec((_BLK, D), lambda i: (i, 0)),
            pl.BlockSpec((_BLK, D), lambda i: (i, 0)),
            pl.BlockSpec((_BLK, 1), lambda i: (i, 0)),
            pl.BlockSpec((D, D), lambda i: (0, 0)),
            pl.BlockSpec((1, D), lambda i: (0, 0)),
        ],
        out_specs=[
            pl.BlockSpec((_BLK, D), lambda i: (i, 0)),
            pl.BlockSpec((_BLK, D), lambda i: (i, 0)),
        ],
        out_shape=[
            jax.ShapeDtypeStruct((N, D), f32),
            jax.ShapeDtypeStruct((N, D), f32),
        ],
    )(acc1, hd1, dis_col, W2, b2)


def _out_body(a_ref, hd_ref, dis_ref, o_ref):
    dis = dis_ref[...]
    o_ref[...] = _elu(a_ref[...] * dis + hd_ref[...])


def _out_call(acc2, hd2, dis_col):
    return pl.pallas_call(
        _out_body,
        grid=(_GRID,),
        in_specs=[
            pl.BlockSpec((_BLK, D), lambda i: (i, 0)),
            pl.BlockSpec((_BLK, D), lambda i: (i, 0)),
            pl.BlockSpec((_BLK, 1), lambda i: (i, 0)),
        ],
        out_specs=pl.BlockSpec((_BLK, D), lambda i: (i, 0)),
        out_shape=jax.ShapeDtypeStruct((N, D), f32),
    )(acc2, hd2, dis_col)


def kernel(x, edge_index, W1, b1, W2, b2):
    src = edge_index[0]
    dst = edge_index[1]
    pad = EPAD - E
    epack = jnp.concatenate(
        [src | (dst << 16), jnp.full((pad,), N << 16, i32)]
    ).reshape(TOTCH, CH)

    deg = _deg_kernel()(epack)
    dis80 = _dis_call(deg)
    dis_col = dis80.reshape(-1)[:N].reshape(N, 1)

    g1, hd1 = _lin1_call(x, W1, b1.reshape(1, D), dis_col)
    acc1 = _agg_kernel()(g1, epack)
    g2, hd2 = _lin2_call(acc1, hd1, dis_col, W2, b2.reshape(1, D))
    acc2 = _agg_kernel()(g2, epack)
    return _out_call(acc2, hd2, dis_col)


# speedup vs baseline: 3.5205x; 3.5205x over previous
import functools

import jax
import jax.numpy as jnp
from jax import lax
from jax.experimental import pallas as pl
from jax.experimental.pallas import tpu as pltpu
from jax.experimental.pallas import tpu_sc as plsc

N = 10000
D = 128
E = 320000
NC = 2
NS = 16
NW = NC * NS
CH = 128
CPT = 80
TOTCH = NW * CPT
EPAD = TOTCH * CH
NPAD = 10112
RPT = NPAD // NS
DEGR = 80
DRPT = 8

f32 = jnp.float32
i32 = jnp.int32


@functools.lru_cache(maxsize=None)
def _mesh():
    return plsc.VectorSubcoreMesh(
        core_axis_name="c", subcore_axis_name="s", num_cores=NC, num_subcores=NS
    )


def _zero_vmem_rows(ref, nrows):
    zero16 = jnp.zeros((16,), f32)

    def zrow(i, c):
        for k in range(8):
            ref[i, pl.ds(k * 16, 16)] = zero16
        return c

    lax.fori_loop(0, nrows, zrow, 0)


def _deg_body(ep_hbm, deg_out, didx, deg1d, deg2d, rowidx, shared_deg):
    cid = lax.axis_index("c")
    sid = lax.axis_index("s")
    wid = cid * NS + sid

    zero16 = jnp.zeros((16,), f32)

    def z1(i, c):
        deg1d[pl.ds(i * 16, 16)] = zero16
        return c

    lax.fori_loop(0, DEGR * D // 16, z1, 0)
    _zero_vmem_rows(deg2d, DEGR)

    @pl.when(sid == 0)
    def _():
        pltpu.sync_copy(deg2d, shared_deg)

    for k in range(DEGR // 16):
        rowidx[0, pl.ds(k * 16, 16)] = lax.iota(i32, 16) + (16 * k)

    pltpu.sync_copy(ep_hbm.at[pl.ds(wid * CPT, CPT)], didx)
    plsc.subcore_barrier()

    ones16 = jnp.ones((16,), f32)

    def jbody(j, c):
        for k in range(CH // 16):
            v = didx[j, pl.ds(k * 16, 16)]
            plsc.addupdate_scatter(
                deg1d, [lax.shift_right_logical(v, 16)], ones16
            )
        return c

    lax.fori_loop(0, CPT, jbody, 0)

    def stage(r, c):
        for k in range(8):
            deg2d[r, pl.ds(k * 16, 16)] = deg1d[pl.ds(r * D + k * 16, 16)]
        return c

    lax.fori_loop(0, DEGR, stage, 0)

    pltpu.sync_copy(deg2d, shared_deg.at[rowidx.at[0]], add=True)
    plsc.subcore_barrier()

    @pl.when(sid < DEGR // DRPT)
    def _():
        r0 = sid * DRPT
        pltpu.sync_copy(
            shared_deg.at[pl.ds(r0, DRPT)], deg_out.at[cid].at[pl.ds(r0, DRPT)]
        )


@functools.lru_cache(maxsize=None)
def _deg_kernel():
    return pl.kernel(
        _deg_body,
        out_type=jax.ShapeDtypeStruct((NC, DEGR, D), f32),
        mesh=_mesh(),
        compiler_params=pltpu.CompilerParams(needs_layout_passes=False),
        scratch_types=[
            pltpu.VMEM((CPT, CH), i32),
            pltpu.VMEM((DEGR * D,), f32),
            pltpu.VMEM((DEGR, D), f32),
            pltpu.VMEM((1, DEGR), i32),
            pltpu.VMEM_SHARED((DEGR, D), f32),
        ],
    )


def _agg_body(g_hbm, ep_hbm, acc_out, eidx, rbuf0, rbuf1, sstage, dstage,
              gsem0, gsem1, shared_acc):
    cid = lax.axis_index("c")
    sid = lax.axis_index("s")
    wid = cid * NS + sid

    _zero_vmem_rows(rbuf0, CH)

    base = sid * RPT
    nfull = RPT // CH
    for b in range(nfull):
        pltpu.sync_copy(rbuf0, shared_acc.at[pl.ds(base + b * CH, CH)])
    rem = RPT - nfull * CH
    pltpu.sync_copy(
        rbuf0.at[pl.ds(0, rem)], shared_acc.at[pl.ds(base + nfull * CH, rem)]
    )

    plsc.subcore_barrier()

    def unpack(j, b):
        for k in range(CH // 16):
            v = eidx[j, pl.ds(k * 16, 16)]
            sstage[b, pl.ds(k * 16, 16)] = lax.bitwise_and(v, 0xFFFF)
            dstage[b, pl.ds(k * 16, 16)] = lax.shift_right_logical(v, 16)

    def run(chunk0, cnt):
        pltpu.sync_copy(
            ep_hbm.at[pl.ds(chunk0, cnt)], eidx.at[pl.ds(0, cnt)]
        )
        unpack(0, 0)
        unpack(1, 1)
        pltpu.async_copy(g_hbm.at[sstage.at[0]], rbuf0, gsem0)
        pltpu.async_copy(g_hbm.at[sstage.at[1]], rbuf1, gsem1)

        def jbody(jj, c):
            j = 2 * jj
            pltpu.make_async_copy(g_hbm.at[sstage.at[0]], rbuf0, gsem0).wait()
            pltpu.sync_copy(rbuf0, shared_acc.at[dstage.at[0]], add=True)
            unpack(j + 2, 0)
            pltpu.async_copy(g_hbm.at[sstage.at[0]], rbuf0, gsem0)
            pltpu.make_async_copy(g_hbm.at[sstage.at[1]], rbuf1, gsem1).wait()
            pltpu.sync_copy(rbuf1, shared_acc.at[dstage.at[1]], add=True)
            unpack(j + 3, 1)
            pltpu.async_copy(g_hbm.at[sstage.at[1]], rbuf1, gsem1)
            return c

        lax.fori_loop(0, cnt // 2 - 1, jbody, 0)

        pltpu.make_async_copy(g_hbm.at[sstage.at[0]], rbuf0, gsem0).wait()
        pltpu.sync_copy(rbuf0, shared_acc.at[dstage.at[0]], add=True)
        pltpu.make_async_copy(g_hbm.at[sstage.at[1]], rbuf1, gsem1).wait()
        pltpu.sync_copy(rbuf1, shared_acc.at[dstage.at[1]], add=True)

    run(wid * CPT, CPT)

    plsc.subcore_barrier()
    pltpu.sync_copy(
        shared_acc.at[pl.ds(base, RPT)], acc_out.at[cid].at[pl.ds(base, RPT)]
    )


@functools.lru_cache(maxsize=None)
def _agg_kernel():
    return pl.kernel(
        _agg_body,
        out_type=jax.ShapeDtypeStruct((NC, NPAD, D), f32),
        mesh=_mesh(),
        compiler_params=pltpu.CompilerParams(needs_layout_passes=False),
        scratch_types=[
            pltpu.VMEM((CPT, CH), i32),
            pltpu.VMEM((CH, D), f32),
            pltpu.VMEM((CH, D), f32),
            pltpu.VMEM((2, CH), i32),
            pltpu.VMEM((2, CH), i32),
            pltpu.SemaphoreType.DMA,
            pltpu.SemaphoreType.DMA,
            pltpu.VMEM_SHARED((NPAD, D), f32),
        ],
    )


_BLK = 400
_GRID = N // _BLK


def _dis_body(deg_ref, dis_ref):
    dis_ref[...] = lax.rsqrt(1.0 + deg_ref[0] + deg_ref[1])


def _dis_call(deg):
    return pl.pallas_call(
        _dis_body,
        out_shape=jax.ShapeDtypeStruct((DEGR, D), f32),
    )(deg)


def _lin1_body(x_ref, w_ref, b_ref, dis_ref, g_ref, hd_ref):
    h = jnp.dot(x_ref[...], w_ref[...], preferred_element_type=f32)
    h = h + b_ref[...]
    dis = dis_ref[...]
    g_ref[...] = h * dis
    hd_ref[...] = h * (dis * dis)


def _lin1_call(x, W1, b1, dis_col):
    return pl.pallas_call(
        _lin1_body,
        grid=(_GRID,),
        in_specs=[
            pl.BlockSpec((_BLK, D), lambda i: (i, 0)),
            pl.BlockSpec((D, D), lambda i: (0, 0)),
            pl.BlockSpec((1, D), lambda i: (0, 0)),
            pl.BlockSpec((_BLK, 1), lambda i: (i, 0)),
        ],
        out_specs=[
            pl.BlockSpec((_BLK, D), lambda i: (i, 0)),
            pl.BlockSpec((_BLK, D), lambda i: (i, 0)),
        ],
        out_shape=[
            jax.ShapeDtypeStruct((N, D), f32),
            jax.ShapeDtypeStruct((N, D), f32),
        ],
    )(x, W1, b1, dis_col)


def _elu(s):
    return jnp.where(s > 0, s, jnp.exp(s) - 1.0)


def _lin2_body(a_ref, hd_ref, dis_ref, w_ref, b_ref, g_ref, hd2_ref):
    dis = dis_ref[...]
    s = (a_ref[0] + a_ref[1]) * dis + hd_ref[...]
    o = _elu(s)
    h2 = jnp.dot(o, w_ref[...], preferred_element_type=f32) + b_ref[...]
    g_ref[...] = h2 * dis
    hd2_ref[...] = h2 * (dis * dis)


def _lin2_call(acc1, hd1, dis_col, W2, b2):
    return pl.pallas_call(
        _lin2_body,
        grid=(_GRID,),
        in_specs=[
            pl.BlockSpec((NC, _BLK, D), lambda i: (0, i, 0)),
            pl.BlockSpec((_BLK, D), lambda i: (i, 0)),
            pl.BlockSpec((_BLK, 1), lambda i: (i, 0)),
            pl.BlockSpec((D, D), lambda i: (0, 0)),
            pl.BlockSpec((1, D), lambda i: (0, 0)),
        ],
        out_specs=[
            pl.BlockSpec((_BLK, D), lambda i: (i, 0)),
            pl.BlockSpec((_BLK, D), lambda i: (i, 0)),
        ],
        out_shape=[
            jax.ShapeDtypeStruct((N, D), f32),
            jax.ShapeDtypeStruct((N, D), f32),
        ],
    )(acc1, hd1, dis_col, W2, b2)


def _out_body(a_ref, hd_ref, dis_ref, o_ref):
    dis = dis_ref[...]
    o_ref[...] = _elu((a_ref[0] + a_ref[1]) * dis + hd_ref[...])


def _out_call(acc2, hd2, dis_col):
    return pl.pallas_call(
        _out_body,
        grid=(_GRID,),
        in_specs=[
            pl.BlockSpec((NC, _BLK, D), lambda i: (0, i, 0)),
            pl.BlockSpec((_BLK, D), lambda i: (i, 0)),
            pl.BlockSpec((_BLK, 1), lambda i: (i, 0)),
        ],
        out_specs=pl.BlockSpec((_BLK, D), lambda i: (i, 0)),
        out_shape=jax.ShapeDtypeStruct((N, D), f32),
    )(acc2, hd2, dis_col)


def kernel(x, edge_index, W1, b1, W2, b2):
    src = edge_index[0]
    dst = edge_index[1]
    pad = EPAD - E
    pidx = jnp.arange(pad, dtype=i32)
    ppack = (pidx % N) | ((N + pidx % (NPAD - N)) << 16)
    epack = jnp.concatenate([src | (dst << 16), ppack]).reshape(TOTCH, CH)

    deg = _deg_kernel()(epack)
    dis80 = _dis_call(deg)
    dis_col = dis80.reshape(-1)[:N].reshape(N, 1)

    g1, hd1 = _lin1_call(x, W1, b1.reshape(1, D), dis_col)
    acc1 = _agg_kernel()(g1, epack)
    g2, hd2 = _lin2_call(acc1, hd1, dis_col, W2, b2.reshape(1, D))
    acc2 = _agg_kernel()(g2, epack)
    return _out_call(acc2, hd2, dis_col)


# speedup vs baseline: 3.5488x; 1.0080x over previous
import functools

import jax
import jax.numpy as jnp
import numpy as np
from jax import lax
from jax.experimental import pallas as pl
from jax.experimental.pallas import tpu as pltpu
from jax.experimental.pallas import tpu_sc as plsc

N = 10000
D = 128
E = 320000
NC = 2
NS = 16
NW = NC * NS
CH = 128
CPT = 80
TOTCH = NW * CPT
EPAD = TOTCH * CH
NPAD = 10112
RPT = NPAD // NS
DEGR = 80
DRPT = 8

f32 = jnp.float32
i32 = jnp.int32

_PPACK = np.asarray(
    (np.arange(EPAD - E) % N) | ((N + np.arange(EPAD - E) % (NPAD - N)) << 16),
    dtype=np.int32,
)


@functools.lru_cache(maxsize=None)
def _mesh():
    return plsc.VectorSubcoreMesh(
        core_axis_name="c", subcore_axis_name="s", num_cores=NC, num_subcores=NS
    )


def _zero_vmem_rows(ref, nrows):
    zero16 = jnp.zeros((16,), f32)

    def zrow(i, c):
        for k in range(8):
            ref[i, pl.ds(k * 16, 16)] = zero16
        return c

    lax.fori_loop(0, nrows, zrow, 0)


def _deg_body(ep_hbm, deg_out, didx, deg1d, deg2d, rowidx, shared_deg):
    cid = lax.axis_index("c")
    sid = lax.axis_index("s")
    wid = cid * NS + sid

    zero16 = jnp.zeros((16,), f32)

    def z1(i, c):
        deg1d[pl.ds(i * 16, 16)] = zero16
        return c

    lax.fori_loop(0, DEGR * D // 16, z1, 0)
    _zero_vmem_rows(deg2d, DEGR)

    @pl.when(sid == 0)
    def _():
        pltpu.sync_copy(deg2d, shared_deg)

    for k in range(DEGR // 16):
        rowidx[0, pl.ds(k * 16, 16)] = lax.iota(i32, 16) + (16 * k)

    pltpu.sync_copy(ep_hbm.at[pl.ds(wid * CPT, CPT)], didx)
    plsc.subcore_barrier()

    ones16 = jnp.ones((16,), f32)

    def jbody(j, c):
        for k in range(CH // 16):
            v = didx[j, pl.ds(k * 16, 16)]
            plsc.addupdate_scatter(
                deg1d, [lax.shift_right_logical(v, 16)], ones16
            )
        return c

    lax.fori_loop(0, CPT, jbody, 0)

    def stage(r, c):
        for k in range(8):
            deg2d[r, pl.ds(k * 16, 16)] = deg1d[pl.ds(r * D + k * 16, 16)]
        return c

    lax.fori_loop(0, DEGR, stage, 0)

    pltpu.sync_copy(deg2d, shared_deg.at[rowidx.at[0]], add=True)
    plsc.subcore_barrier()

    @pl.when(sid < DEGR // DRPT)
    def _():
        r0 = sid * DRPT
        pltpu.sync_copy(
            shared_deg.at[pl.ds(r0, DRPT)], deg_out.at[cid].at[pl.ds(r0, DRPT)]
        )


@functools.lru_cache(maxsize=None)
def _deg_kernel():
    return pl.kernel(
        _deg_body,
        out_type=jax.ShapeDtypeStruct((NC, DEGR, D), f32),
        mesh=_mesh(),
        compiler_params=pltpu.CompilerParams(needs_layout_passes=False),
        scratch_types=[
            pltpu.VMEM((CPT, CH), i32),
            pltpu.VMEM((DEGR * D,), f32),
            pltpu.VMEM((DEGR, D), f32),
            pltpu.VMEM((1, DEGR), i32),
            pltpu.VMEM_SHARED((DEGR, D), f32),
        ],
    )


def _agg_body(g_hbm, ep_hbm, acc_out, eidx, rbuf0, rbuf1, sstage, dstage,
              gsem0, gsem1, shared_acc):
    cid = lax.axis_index("c")
    sid = lax.axis_index("s")
    wid = cid * NS + sid

    _zero_vmem_rows(rbuf0, CH)

    base = sid * RPT
    nfull = RPT // CH
    for b in range(nfull):
        pltpu.sync_copy(rbuf0, shared_acc.at[pl.ds(base + b * CH, CH)])
    rem = RPT - nfull * CH
    pltpu.sync_copy(
        rbuf0.at[pl.ds(0, rem)], shared_acc.at[pl.ds(base + nfull * CH, rem)]
    )

    plsc.subcore_barrier()

    def unpack(j, b):
        for k in range(CH // 16):
            v = eidx[j, pl.ds(k * 16, 16)]
            sstage[b, pl.ds(k * 16, 16)] = lax.bitwise_and(v, 0xFFFF)
            dstage[b, pl.ds(k * 16, 16)] = lax.shift_right_logical(v, 16)

    def run(chunk0, cnt):
        pltpu.sync_copy(
            ep_hbm.at[pl.ds(chunk0, cnt)], eidx.at[pl.ds(0, cnt)]
        )
        unpack(0, 0)
        unpack(1, 1)
        pltpu.async_copy(g_hbm.at[sstage.at[0]], rbuf0, gsem0)
        pltpu.async_copy(g_hbm.at[sstage.at[1]], rbuf1, gsem1)

        def jbody(jj, c):
            j = 2 * jj
            pltpu.make_async_copy(g_hbm.at[sstage.at[0]], rbuf0, gsem0).wait()
            pltpu.sync_copy(rbuf0, shared_acc.at[dstage.at[0]], add=True)
            unpack(j + 2, 0)
            pltpu.async_copy(g_hbm.at[sstage.at[0]], rbuf0, gsem0)
            pltpu.make_async_copy(g_hbm.at[sstage.at[1]], rbuf1, gsem1).wait()
            pltpu.sync_copy(rbuf1, shared_acc.at[dstage.at[1]], add=True)
            unpack(j + 3, 1)
            pltpu.async_copy(g_hbm.at[sstage.at[1]], rbuf1, gsem1)
            return c

        lax.fori_loop(0, cnt // 2 - 1, jbody, 0)

        pltpu.make_async_copy(g_hbm.at[sstage.at[0]], rbuf0, gsem0).wait()
        pltpu.sync_copy(rbuf0, shared_acc.at[dstage.at[0]], add=True)
        pltpu.make_async_copy(g_hbm.at[sstage.at[1]], rbuf1, gsem1).wait()
        pltpu.sync_copy(rbuf1, shared_acc.at[dstage.at[1]], add=True)

    run(wid * CPT, CPT)

    plsc.subcore_barrier()
    pltpu.sync_copy(
        shared_acc.at[pl.ds(base, RPT)], acc_out.at[cid].at[pl.ds(base, RPT)]
    )


@functools.lru_cache(maxsize=None)
def _agg_kernel():
    return pl.kernel(
        _agg_body,
        out_type=jax.ShapeDtypeStruct((NC, NPAD, D), f32),
        mesh=_mesh(),
        compiler_params=pltpu.CompilerParams(needs_layout_passes=False),
        scratch_types=[
            pltpu.VMEM((CPT, CH), i32),
            pltpu.VMEM((CH, D), f32),
            pltpu.VMEM((CH, D), f32),
            pltpu.VMEM((2, CH), i32),
            pltpu.VMEM((2, CH), i32),
            pltpu.SemaphoreType.DMA,
            pltpu.SemaphoreType.DMA,
            pltpu.VMEM_SHARED((NPAD, D), f32),
        ],
    )


_BLK = 400
_GRID = N // _BLK


def _dis_body(deg_ref, dis_ref):
    dis_ref[...] = lax.rsqrt(1.0 + deg_ref[0] + deg_ref[1])


def _dis_call(deg):
    return pl.pallas_call(
        _dis_body,
        out_shape=jax.ShapeDtypeStruct((DEGR, D), f32),
    )(deg)


def _lin1_body(x_ref, w_ref, b_ref, dis_ref, g_ref):
    h = jnp.dot(x_ref[...], w_ref[...], preferred_element_type=f32)
    h = h + b_ref[...]
    g_ref[...] = h * dis_ref[...]


def _lin1_call(x, W1, b1, dis_col):
    return pl.pallas_call(
        _lin1_body,
        grid=(_GRID,),
        in_specs=[
            pl.BlockSpec((_BLK, D), lambda i: (i, 0)),
            pl.BlockSpec((D, D), lambda i: (0, 0)),
            pl.BlockSpec((1, D), lambda i: (0, 0)),
            pl.BlockSpec((_BLK, 1), lambda i: (i, 0)),
        ],
        out_specs=pl.BlockSpec((_BLK, D), lambda i: (i, 0)),
        out_shape=jax.ShapeDtypeStruct((N, D), f32),
    )(x, W1, b1, dis_col)


def _elu(s):
    return jnp.where(s > 0, s, jnp.exp(s) - 1.0)


def _lin2_body(a_ref, g_ref_in, dis_ref, w_ref, b_ref, g_ref):
    dis = dis_ref[...]
    o = _elu((a_ref[0] + a_ref[1] + g_ref_in[...]) * dis)
    h2 = jnp.dot(o, w_ref[...], preferred_element_type=f32) + b_ref[...]
    g_ref[...] = h2 * dis


def _lin2_call(acc1, g1, dis_col, W2, b2):
    return pl.pallas_call(
        _lin2_body,
        grid=(_GRID,),
        in_specs=[
            pl.BlockSpec((NC, _BLK, D), lambda i: (0, i, 0)),
            pl.BlockSpec((_BLK, D), lambda i: (i, 0)),
            pl.BlockSpec((_BLK, 1), lambda i: (i, 0)),
            pl.BlockSpec((D, D), lambda i: (0, 0)),
            pl.BlockSpec((1, D), lambda i: (0, 0)),
        ],
        out_specs=pl.BlockSpec((_BLK, D), lambda i: (i, 0)),
        out_shape=jax.ShapeDtypeStruct((N, D), f32),
    )(acc1, g1, dis_col, W2, b2)


def _out_body(a_ref, g_ref_in, dis_ref, o_ref):
    o_ref[...] = _elu((a_ref[0] + a_ref[1] + g_ref_in[...]) * dis_ref[...])


def _out_call(acc2, g2, dis_col):
    return pl.pallas_call(
        _out_body,
        grid=(_GRID,),
        in_specs=[
            pl.BlockSpec((NC, _BLK, D), lambda i: (0, i, 0)),
            pl.BlockSpec((_BLK, D), lambda i: (i, 0)),
            pl.BlockSpec((_BLK, 1), lambda i: (i, 0)),
        ],
        out_specs=pl.BlockSpec((_BLK, D), lambda i: (i, 0)),
        out_shape=jax.ShapeDtypeStruct((N, D), f32),
    )(acc2, g2, dis_col)


def kernel(x, edge_index, W1, b1, W2, b2):
    src = edge_index[0]
    dst = edge_index[1]
    epack = jnp.concatenate([src | (dst << 16), _PPACK]).reshape(TOTCH, CH)

    deg = _deg_kernel()(epack)
    dis80 = _dis_call(deg)
    dis_col = dis80.reshape(-1)[:N].reshape(N, 1)

    g1 = _lin1_call(x, W1, b1.reshape(1, D), dis_col)
    acc1 = _agg_kernel()(g1, epack)
    g2 = _lin2_call(acc1, g1, dis_col, W2, b2.reshape(1, D))
    acc2 = _agg_kernel()(g2, epack)
    return _out_call(acc2, g2, dis_col)
